# Initial kernel scaffold; baseline (speedup 1.0000x reference)
import jax
import jax.numpy as jnp
from jax.experimental import pallas as pl


def kernel(h, bond_atoms, angle_atoms, torsion_atoms, one_four_atoms, nonbonded_atoms, W1_atom, b1_atom, W2_atom, b2_atom, W1_bond, b1_bond, W2_bond, b2_bond, W1_angle, b1_angle, W2_angle, b2_angle, W1_torsion, b1_torsion, W2_torsion, b2_torsion):
    raise NotImplementedError("write your pallas kernel here")


# speedup vs baseline: 9.9564x; 9.9564x over previous
import functools

import jax
import jax.numpy as jnp
from jax import lax
from jax.experimental import pallas as pl
from jax.experimental.pallas import tpu as pltpu
from jax.experimental.pallas import tpu_sc as plsc

NA = 50000
D = 128
RU = 32

NC = 2
NS = 16
NW = NC * NS

BOND = (50000, 2, 1600, 400, 4)
ANGLE = (60000, 3, 1920, 480, 4)
TORSION = (70000, 4, 2240, 560, 4)
P14 = (100000, 3200, 640, 5)
PNB = (200000, 6400, 640, 10)
PBLK = 640

_mesh = plsc.VectorSubcoreMesh(
    core_axis_name="c", subcore_axis_name="s", num_cores=NC, num_subcores=NS)


def _worker_id():
  return lax.axis_index("s") * NC + lax.axis_index("c")


def _stage_a_body(h_ref, w1_ref, b1_ref, w2_ref, b2_ref,
                  zb_ref, za_ref, zt_ref, ke_ref):
  z = jnp.dot(h_ref[...], w1_ref[...], preferred_element_type=jnp.float32)
  zb_ref[...] = z[:, RU:2 * RU]
  za_ref[...] = z[:, 2 * RU:3 * RU]
  zt_ref[...] = z[:, 3 * RU:4 * RU]
  t = jnp.tanh(z[:, :RU] + b1_ref[...])
  ke = jnp.abs(jnp.dot(t, w2_ref[...], preferred_element_type=jnp.float32)
               + b2_ref[...])
  sq = jnp.sqrt(ke[:, 1:2])
  ke_ref[...] = jnp.concatenate([ke, sq, sq], axis=1)


def _stage_a(h, w1cat, b1a, w2a, b2a):
  rb = 1000
  grid = (NA // rb,)
  return pl.pallas_call(
      _stage_a_body,
      grid=grid,
      in_specs=[
          pl.BlockSpec((rb, D), lambda i: (i, 0)),
          pl.BlockSpec((D, 4 * RU), lambda i: (0, 0)),
          pl.BlockSpec((1, RU), lambda i: (0, 0)),
          pl.BlockSpec((RU, 2), lambda i: (0, 0)),
          pl.BlockSpec((1, 2), lambda i: (0, 0)),
      ],
      out_specs=[
          pl.BlockSpec((rb, RU), lambda i: (i, 0)),
          pl.BlockSpec((rb, RU), lambda i: (i, 0)),
          pl.BlockSpec((rb, RU), lambda i: (i, 0)),
          pl.BlockSpec((rb, 4), lambda i: (i, 0)),
      ],
      out_shape=[
          jax.ShapeDtypeStruct((NA, RU), jnp.float32),
          jax.ShapeDtypeStruct((NA, RU), jnp.float32),
          jax.ShapeDtypeStruct((NA, RU), jnp.float32),
          jax.ShapeDtypeStruct((NA, 4), jnp.float32),
      ],
  )(h, w1cat, b1a, w2a, b2a)


def _accum_rows(acc, tmp, nrows):
  def body(r, _):
    acc[r, pl.ds(0, 16)] = acc[r, pl.ds(0, 16)] + tmp[r, pl.ds(0, 16)]
    acc[r, pl.ds(16, 16)] = acc[r, pl.ds(16, 16)] + tmp[r, pl.ds(16, 16)]
    return 0
  lax.fori_loop(0, nrows, body, 0)


def _gather_relation(z_hbm, idx_hbms, out_hbm, spec, idx_v, acc_v, tmp_v, sem):
  _, arity, chunk, blk, nblk = spec
  base = _worker_id() * chunk
  for b in range(nblk):
    off = pl.multiple_of(base + b * blk, 8)
    pltpu.sync_copy(idx_hbms[0].at[pl.ds(off, blk)], idx_v)
    pltpu.async_copy(z_hbm.at[idx_v], acc_v, sem).wait()
    for a in range(1, arity):
      pltpu.sync_copy(idx_hbms[a].at[pl.ds(off, blk)], idx_v)
      pltpu.async_copy(z_hbm.at[idx_v], tmp_v, sem).wait()
      _accum_rows(acc_v, tmp_v, blk)
    pltpu.sync_copy(acc_v, out_hbm.at[pl.ds(off, blk)])


def _relations_body(zb, za, zt, b0, b1_, a0, a1_, a2_, t0, t1_, t2_, t3_,
                    sb, sa, st,
                    bi_v, bacc_v, btmp_v, ai_v, aacc_v, atmp_v,
                    ti_v, tacc_v, ttmp_v, sem):
  _gather_relation(zb, (b0, b1_), sb, BOND, bi_v, bacc_v, btmp_v, sem)
  _gather_relation(za, (a0, a1_, a2_), sa, ANGLE, ai_v, aacc_v, atmp_v, sem)
  _gather_relation(zt, (t0, t1_, t2_, t3_), st, TORSION, ti_v, tacc_v, ttmp_v,
                   sem)


def _relations(zb, za, zt, idx_cols):
  out_type = [
      jax.ShapeDtypeStruct((BOND[2] * NW, RU), jnp.float32),
      jax.ShapeDtypeStruct((ANGLE[2] * NW, RU), jnp.float32),
      jax.ShapeDtypeStruct((TORSION[2] * NW, RU), jnp.float32),
  ]
  scratch = [
      pltpu.VMEM((BOND[3],), jnp.int32),
      pltpu.VMEM((BOND[3], RU), jnp.float32),
      pltpu.VMEM((BOND[3], RU), jnp.float32),
      pltpu.VMEM((ANGLE[3],), jnp.int32),
      pltpu.VMEM((ANGLE[3], RU), jnp.float32),
      pltpu.VMEM((ANGLE[3], RU), jnp.float32),
      pltpu.VMEM((TORSION[3],), jnp.int32),
      pltpu.VMEM((TORSION[3], RU), jnp.float32),
      pltpu.VMEM((TORSION[3], RU), jnp.float32),
      pltpu.SemaphoreType.DMA,
  ]
  fn = pl.kernel(_relations_body, out_type=out_type, mesh=_mesh,
                 scratch_types=scratch,
                 compiler_params=pltpu.CompilerParams(
                     use_tc_tiling_on_sc=False))
  return fn(zb, za, zt, *idx_cols)


def _pair_loop(i0_hbm, i1_hbm, sig_hbm, eps_hbm, spec,
               ktbl_v, stbl_v, ia_v, ib_v, so_v, eo_v, sem):
  del sem
  _, chunk, blk, nblk = spec
  base = _worker_id() * chunk

  def group(si, _):
    o = pl.multiple_of(si * 16, 16)
    ia = ia_v[pl.ds(o, 16)]
    ib = ib_v[pl.ds(o, 16)]
    ka = plsc.load_gather(ktbl_v, [ia])
    kb = plsc.load_gather(ktbl_v, [ib])
    so_v[pl.ds(o, 16)] = (ka + kb) * 0.5
    ea = plsc.load_gather(stbl_v, [ia])
    eb = plsc.load_gather(stbl_v, [ib])
    eo_v[pl.ds(o, 16)] = ea * eb
    return 0

  for b in range(nblk):
    off = pl.multiple_of(base + b * blk, 8)
    pltpu.sync_copy(i0_hbm.at[pl.ds(off, blk)], ia_v)
    pltpu.sync_copy(i1_hbm.at[pl.ds(off, blk)], ib_v)
    lax.fori_loop(0, blk // 16, group, 0)
    pltpu.sync_copy(so_v, sig_hbm.at[pl.ds(off, blk)])
    pltpu.sync_copy(eo_v, eps_hbm.at[pl.ds(off, blk)])


def _pairs_body(ktbl, stbl, p140, p141, pnb0, pnb1,
                sig14, eps14, signb, epsnb,
                ktbl_v, stbl_v, ia_v, ib_v, so_v, eo_v, sem):
  pltpu.sync_copy(ktbl, ktbl_v)
  pltpu.sync_copy(stbl, stbl_v)
  _pair_loop(p140, p141, sig14, eps14, P14,
             ktbl_v, stbl_v, ia_v, ib_v, so_v, eo_v, sem)
  _pair_loop(pnb0, pnb1, signb, epsnb, PNB,
             ktbl_v, stbl_v, ia_v, ib_v, so_v, eo_v, sem)


def _pairs(ktbl, stbl, p140, p141, pnb0, pnb1):
  out_type = [
      jax.ShapeDtypeStruct((P14[1] * NW,), jnp.float32),
      jax.ShapeDtypeStruct((P14[1] * NW,), jnp.float32),
      jax.ShapeDtypeStruct((PNB[1] * NW,), jnp.float32),
      jax.ShapeDtypeStruct((PNB[1] * NW,), jnp.float32),
  ]
  scratch = [
      pltpu.VMEM((NA,), jnp.float32),
      pltpu.VMEM((NA,), jnp.float32),
      pltpu.VMEM((PBLK,), jnp.int32),
      pltpu.VMEM((PBLK,), jnp.int32),
      pltpu.VMEM((PBLK,), jnp.float32),
      pltpu.VMEM((PBLK,), jnp.float32),
      pltpu.SemaphoreType.DMA,
  ]
  fn = pl.kernel(_pairs_body, out_type=out_type, mesh=_mesh,
                 scratch_types=scratch,
                 compiler_params=pltpu.CompilerParams(
                     use_tc_tiling_on_sc=False, needs_layout_passes=False))
  return fn(ktbl, stbl, p140, p141, pnb0, pnb1)


def _stage_c_body(s_ref, b1_ref, w2_ref, b2_ref, out_ref):
  t = jnp.tanh(s_ref[...] + b1_ref[...])
  out_ref[...] = jnp.abs(
      jnp.dot(t, w2_ref[...], preferred_element_type=jnp.float32)
      + b2_ref[...])


def _stage_c(s, b1, w2, b2):
  npad = s.shape[0]
  rb = 512
  return pl.pallas_call(
      _stage_c_body,
      grid=(npad // rb,),
      in_specs=[
          pl.BlockSpec((rb, RU), lambda i: (i, 0)),
          pl.BlockSpec((1, RU), lambda i: (0, 0)),
          pl.BlockSpec((RU, 2), lambda i: (0, 0)),
          pl.BlockSpec((1, 2), lambda i: (0, 0)),
      ],
      out_specs=pl.BlockSpec((rb, 2), lambda i: (i, 0)),
      out_shape=jax.ShapeDtypeStruct((npad, 2), jnp.float32),
  )(s, b1, w2, b2)


def _pad_cols(atoms, npad):
  n = atoms.shape[0]
  return [jnp.pad(atoms[:, a], (0, npad - n)) for a in range(atoms.shape[1])]


def kernel(h, bond_atoms, angle_atoms, torsion_atoms, one_four_atoms,
           nonbonded_atoms,
           W1_atom, b1_atom, W2_atom, b2_atom,
           W1_bond, b1_bond, W2_bond, b2_bond,
           W1_angle, b1_angle, W2_angle, b2_angle,
           W1_torsion, b1_torsion, W2_torsion, b2_torsion):
  w1cat = jnp.concatenate([W1_atom, W1_bond, W1_angle, W1_torsion], axis=1)
  zb, za, zt, ke = _stage_a(h, w1cat, b1_atom.reshape(1, RU),
                            W2_atom, b2_atom.reshape(1, 2))
  k_atom = ke[:, 0]
  eq_atom = ke[:, 1]
  sq_eq = ke[:, 2]

  idx_cols = (_pad_cols(bond_atoms, BOND[2] * NW)
              + _pad_cols(angle_atoms, ANGLE[2] * NW)
              + _pad_cols(torsion_atoms, TORSION[2] * NW))
  sb, sa, st = _relations(zb, za, zt, idx_cols)

  p14 = _pad_cols(one_four_atoms, P14[1] * NW)
  pnb = _pad_cols(nonbonded_atoms, PNB[1] * NW)
  sig14, eps14, signb, epsnb = _pairs(k_atom, sq_eq, p14[0], p14[1],
                                      pnb[0], pnb[1])

  rb_ = _stage_c(sb, b1_bond.reshape(1, RU), W2_bond, b2_bond.reshape(1, 2))
  ra_ = _stage_c(sa, b1_angle.reshape(1, RU), W2_angle, b2_angle.reshape(1, 2))
  rt_ = _stage_c(st, b1_torsion.reshape(1, RU), W2_torsion,
                 b2_torsion.reshape(1, 2))

  return jnp.concatenate([
      k_atom, eq_atom,
      rb_[:BOND[0], 0], rb_[:BOND[0], 1],
      ra_[:ANGLE[0], 0], ra_[:ANGLE[0], 1],
      rt_[:TORSION[0], 0], rt_[:TORSION[0], 1],
      sig14[:P14[0]], eps14[:P14[0]],
      signb[:PNB[0]], epsnb[:PNB[0]],
  ])
